# baseline (speedup 1.0000x reference)
import functools

import jax
import jax.numpy as jnp
from jax import lax
from jax.experimental import pallas as pl
from jax.experimental.pallas import tpu as pltpu
from jax.experimental.pallas import tpu_sc as plsc

N_NODES = 10000
N_EDGES = 320000
D_IN = 128
D_HID = 256
N_CLS = 40
EPS = 1e-5

NC = 2
NS = 16
DEG_W = 8

N_PAD = 10112
E_PAD = 327680
E_PER_SUB = E_PAD // NS
ROWS_PER_SUB = N_PAD // NS

NIDX = 8
LI = 4


def _sc_agg_body(with_deg, dc, blk, nbuf, look, *refs):
    if with_deg:
        (eidx, tab0, tab1, zrs, zrs_deg, ones,
         out0, out1, dout, *rest) = refs
    else:
        (eidx, tab0, tab1, zrs, out0, out1, *rest) = refs
    ibuf = rest[:NIDX]
    rows = rest[NIDX:NIDX + nbuf]
    acc = rest[NIDX + nbuf]
    isem = rest[NIDX + nbuf + 1:NIDX + nbuf + 1 + NIDX]
    gsem = rest[NIDX + nbuf + 1 + NIDX:NIDX + nbuf + 1 + NIDX + nbuf]
    ssem = rest[NIDX + nbuf + 1 + NIDX + nbuf:
                NIDX + nbuf + 1 + NIDX + 2 * nbuf]
    if with_deg:
        degsem, ones_v, deg_acc = rest[NIDX + nbuf + 1 + NIDX + 2 * nbuf:]

    n_loc = E_PER_SUB // blk

    c = lax.axis_index("c")
    s = lax.axis_index("s")
    rbase = s * ROWS_PER_SUB
    stripe = pl.ds(rbase, ROWS_PER_SUB)

    pltpu.sync_copy(zrs.at[stripe], acc.at[stripe])
    if with_deg:
        pltpu.sync_copy(zrs_deg.at[stripe], deg_acc.at[stripe])
        pltpu.sync_copy(ones, ones_v)
    plsc.subcore_barrier()

    def iissue(k, bi):
        pltpu.async_copy(eidx.at[s, pl.ds(2 * k, 2)], ibuf[bi], isem[bi])

    def iwait(k, bi):
        pltpu.make_async_copy(eidx.at[s, pl.ds(2 * k, 2)], ibuf[bi],
                              isem[bi]).wait()

    def gissue(k, bi, b):
        @pl.when(c == 0)
        def _():
            pltpu.async_copy(tab0.at[ibuf[bi].at[0]], rows[b], gsem[b])

        @pl.when(c == 1)
        def _():
            pltpu.async_copy(tab1.at[ibuf[bi].at[0]], rows[b], gsem[b])

    def gwait(bi, b):
        pltpu.make_async_copy(tab0.at[ibuf[bi].at[0]], rows[b],
                              gsem[b]).wait()

    def sissue(bi, b):
        pltpu.async_copy(rows[b], acc.at[ibuf[bi].at[1]], ssem[b], add=True)

    def swait(bi, b):
        pltpu.make_async_copy(rows[b], acc.at[ibuf[bi].at[1]],
                              ssem[b]).wait()

    def dissue(bi):
        pltpu.async_copy(ones_v, deg_acc.at[ibuf[bi].at[1]], degsem,
                         add=True)

    def dwait(bi):
        pltpu.make_async_copy(ones_v, deg_acc.at[ibuf[bi].at[1]],
                              degsem).wait()

    for k in range(LI):
        iissue(k, k)
    for k in range(look):
        iwait(k, k)
        gissue(k, k, k)

    def outer(k8, carry):
        for u in range(NIDX):
            k = k8 * NIDX + u
            b = u % nbuf
            bi = u

            @pl.when(k + LI < n_loc)
            def _(k=k, bi2=(u + LI) % NIDX):
                iissue(k + LI, bi2)

            gwait(bi, b)
            sissue(bi, b)
            if with_deg:
                dissue(bi)

                @pl.when(k >= look)
                def _(bi2=(u - look) % NIDX):
                    dwait(bi2)

            bg = (b + look) % nbuf

            @pl.when(k >= look)
            def _(bi2=(u - look) % NIDX, bg=bg):
                swait(bi2, bg)

            @pl.when(k + look < n_loc)
            def _(k=k, bi2=(u + look) % NIDX, bg=bg):
                iwait(k + look, bi2)
                gissue(k + look, bi2, bg)
        return carry

    lax.fori_loop(0, n_loc // NIDX, outer, 0)
    for k in range(n_loc - look, n_loc):
        swait(k % NIDX, k % nbuf)
        if with_deg:
            dwait(k % NIDX)
    plsc.subcore_barrier()

    @pl.when(c == 0)
    def _():
        pltpu.sync_copy(acc.at[stripe], out0.at[stripe])
        if with_deg:
            pltpu.sync_copy(deg_acc.at[stripe], dout.at[stripe])

    @pl.when(c == 1)
    def _():
        pltpu.sync_copy(acc.at[stripe], out1.at[stripe])


def _make_sc_agg(dc, blk, nbuf=4, look=2, with_deg=False):
    mesh = plsc.VectorSubcoreMesh(core_axis_name="c", subcore_axis_name="s")
    out_type = [jax.ShapeDtypeStruct((N_PAD, dc), jnp.float32),
                jax.ShapeDtypeStruct((N_PAD, dc), jnp.float32)]
    scratch = (
        [pltpu.VMEM((2, blk), jnp.int32)] * NIDX
        + [pltpu.VMEM((blk, dc), jnp.float32)] * nbuf
        + [pltpu.VMEM_SHARED((N_PAD, dc), jnp.float32)]
        + [pltpu.SemaphoreType.DMA] * (NIDX + 2 * nbuf)
    )
    if with_deg:
        out_type += [jax.ShapeDtypeStruct((N_PAD, DEG_W), jnp.float32)]
        scratch += [
            pltpu.SemaphoreType.DMA,
            pltpu.VMEM((blk, DEG_W), jnp.float32),
            pltpu.VMEM_SHARED((N_PAD, DEG_W), jnp.float32),
        ]
    return pl.kernel(
        functools.partial(_sc_agg_body, with_deg, dc, blk, nbuf, look),
        out_type=out_type,
        mesh=mesh,
        scratch_types=scratch,
        compiler_params=pltpu.CompilerParams(use_tc_tiling_on_sc=False),
    )


def _pack_eidx(src_p, dst_p, blk):
    nb = E_PER_SUB // blk
    e = jnp.stack([src_p.reshape(NS, nb, blk),
                   dst_p.reshape(NS, nb, blk)], axis=2)
    return e.reshape(NS, 2 * nb, blk)


def _rdeg(deg_ref):
    return 1.0 / jnp.maximum(deg_ref[:N_NODES, 0:1], 1.0)


def _bn_relu(h, g_ref, be_ref):
    mu = jnp.mean(h, axis=0, keepdims=True)
    var = jnp.mean((h - mu) * (h - mu), axis=0, keepdims=True)
    h = g_ref[...] * (h - mu) * lax.rsqrt(var + EPS) + be_ref[...]
    return jnp.maximum(h, 0.0)


def _tcself1_body(x_ref, ws_ref, o_ref):
    o_ref[...] = jnp.dot(x_ref[...], ws_ref[...],
                         preferred_element_type=jnp.float32)


def _tcself2_body(x0_ref, x1_ref, ws_ref, o_ref):
    f32 = jnp.float32
    o = jnp.dot(x0_ref[...], ws_ref[0], preferred_element_type=f32)
    o += jnp.dot(x1_ref[...], ws_ref[1], preferred_element_type=f32)
    o_ref[...] = o


def _tc1_body(xw_ref, a0_ref, a1_ref, d_ref,
              wn_ref, b_ref, g_ref, be_ref,
              h0_ref, h1_ref):
    rdeg = _rdeg(d_ref)
    f32 = jnp.float32
    h = xw_ref[...] + b_ref[...]
    h += jnp.dot(a0_ref[:N_NODES] * rdeg, wn_ref[0], preferred_element_type=f32)
    h += jnp.dot(a1_ref[:N_NODES] * rdeg, wn_ref[1], preferred_element_type=f32)
    h = _bn_relu(h, g_ref, be_ref)
    h0_ref[...] = h[:, :D_HID // 2]
    h1_ref[...] = h[:, D_HID // 2:]


def _tc2_body(xw_ref, a0_ref, a1_ref, d_ref,
              wn_ref, b_ref, g_ref, be_ref, wn3_ref,
              h0_ref, h1_ref, p_ref):
    rdeg = _rdeg(d_ref)
    f32 = jnp.float32
    h = xw_ref[...] + b_ref[...]
    h += jnp.dot(a0_ref[:N_NODES] * rdeg, wn_ref[0], preferred_element_type=f32)
    h += jnp.dot(a1_ref[:N_NODES] * rdeg, wn_ref[1], preferred_element_type=f32)
    h = _bn_relu(h, g_ref, be_ref)
    h0_ref[...] = h[:, :D_HID // 2]
    h1_ref[...] = h[:, D_HID // 2:]
    p_ref[...] = jnp.dot(h, wn3_ref[...], preferred_element_type=f32)


def _tc3_body(xw_ref, a0_ref, a1_ref, d_ref, b_ref, out_ref):
    rdeg = _rdeg(d_ref)
    mean = jnp.concatenate(
        [a0_ref[:N_NODES], a1_ref[:N_NODES, :N_CLS - 32]], axis=1) * rdeg
    logits = xw_ref[...] + mean + b_ref[...]
    m = jnp.max(logits, axis=1, keepdims=True)
    sh = logits - m
    lse = jnp.log(jnp.sum(jnp.exp(sh), axis=1, keepdims=True))
    out_ref[...] = sh - lse


def kernel(x, edge_index,
           W_self1, W_neigh1, b1, gamma1, beta1,
           W_self2, W_neigh2, b2, gamma2, beta2,
           W_self3, W_neigh3, b3):
    f32 = jnp.float32
    src = edge_index[0].astype(jnp.int32)
    dst = edge_index[1].astype(jnp.int32)
    pad = E_PAD - N_EDGES
    src_p = jnp.concatenate([src, jnp.zeros((pad,), jnp.int32)])
    pad_dst = N_NODES + jnp.arange(pad, dtype=jnp.int32) % (N_PAD - N_NODES)
    dst_p = jnp.concatenate([dst, pad_dst])
    eidx128 = _pack_eidx(src_p, dst_p, 128)

    x0 = x[:, :D_IN // 2]
    x1 = x[:, D_IN // 2:]
    z64 = jnp.zeros((N_PAD, 64), f32)
    z128 = jnp.zeros((N_PAD, 128), f32)
    z32 = jnp.zeros((N_PAD, 32), f32)
    zdeg = jnp.zeros((N_PAD, DEG_W), f32)
    ones = jnp.ones((128, DEG_W), f32)

    agg1_0, agg1_1, deg = _make_sc_agg(64, 128, with_deg=True)(
        eidx128, x0, x1, z64, zdeg, ones)
    xw1 = pl.pallas_call(
        _tcself1_body,
        out_shape=jax.ShapeDtypeStruct((N_NODES, D_HID), f32),
    )(x, W_self1)

    wn1 = W_neigh1.reshape(2, 64, D_HID)
    b1r, g1r, be1r = b1.reshape(1, -1), gamma1.reshape(1, -1), beta1.reshape(1, -1)
    h1_0, h1_1 = pl.pallas_call(
        _tc1_body,
        out_shape=[jax.ShapeDtypeStruct((N_NODES, D_HID // 2), f32)] * 2,
    )(xw1, agg1_0, agg1_1, deg, wn1, b1r, g1r, be1r)

    agg2_0, agg2_1 = _make_sc_agg(128, 128, nbuf=2, look=1)(
        eidx128, h1_0, h1_1, z128)
    ws2 = W_self2.reshape(2, 128, D_HID)
    xw2 = pl.pallas_call(
        _tcself2_body,
        out_shape=jax.ShapeDtypeStruct((N_NODES, D_HID), f32),
    )(h1_0, h1_1, ws2)

    wn2 = W_neigh2.reshape(2, 128, D_HID)
    wn3p = jnp.pad(W_neigh3, ((0, 0), (0, 64 - N_CLS)))
    b2r, g2r, be2r = b2.reshape(1, -1), gamma2.reshape(1, -1), beta2.reshape(1, -1)
    h2_0, h2_1, p3 = pl.pallas_call(
        _tc2_body,
        out_shape=[jax.ShapeDtypeStruct((N_NODES, D_HID // 2), f32)] * 2
        + [jax.ShapeDtypeStruct((N_NODES, 64), f32)],
    )(xw2, agg2_0, agg2_1, deg, wn2, b2r, g2r, be2r, wn3p)
    p3_0 = p3[:, :32]
    p3_1 = p3[:, 32:]

    agg3_0, agg3_1 = _make_sc_agg(32, 128)(eidx128, p3_0, p3_1, z32)
    ws3 = W_self3.reshape(2, 128, N_CLS)
    xw3 = pl.pallas_call(
        _tcself2_body,
        out_shape=jax.ShapeDtypeStruct((N_NODES, N_CLS), f32),
    )(h2_0, h2_1, ws3)

    b3r = b3.reshape(1, -1)
    out = pl.pallas_call(
        _tc3_body,
        out_shape=jax.ShapeDtypeStruct((N_NODES, N_CLS), f32),
    )(xw3, agg3_0, agg3_1, deg, b3r)
    return out


# speedup vs baseline: 1.5118x; 1.5118x over previous
import functools

import jax
import jax.numpy as jnp
from jax import lax
from jax.experimental import pallas as pl
from jax.experimental.pallas import tpu as pltpu
from jax.experimental.pallas import tpu_sc as plsc

N_NODES = 10000
N_EDGES = 320000
D_IN = 128
D_HID = 256
N_CLS = 40
EPS = 1e-5

NC = 2
NS = 16
DEG_W = 8

N_PAD = 10112
E_PAD = 327680
E_PER_SUB = E_PAD // NS
ROWS_PER_SUB = N_PAD // NS

NIDX = 8
LI = 4


def _sc_agg_body(with_deg, dc, blk, nbuf, look, *refs):
    if with_deg:
        (eidx, tab0, tab1, zrs, zrs_deg, ones,
         out0, out1, dout, *rest) = refs
    else:
        (eidx, tab0, tab1, zrs, out0, out1, *rest) = refs
    ibuf = rest[:NIDX]
    rows = rest[NIDX:NIDX + nbuf]
    acc = rest[NIDX + nbuf]
    isem = rest[NIDX + nbuf + 1:NIDX + nbuf + 1 + NIDX]
    gsem = rest[NIDX + nbuf + 1 + NIDX:NIDX + nbuf + 1 + NIDX + nbuf]
    ssem = rest[NIDX + nbuf + 1 + NIDX + nbuf:
                NIDX + nbuf + 1 + NIDX + 2 * nbuf]
    if with_deg:
        degsem, ones_v, deg_acc = rest[NIDX + nbuf + 1 + NIDX + 2 * nbuf:]

    n_loc = E_PER_SUB // blk

    c = lax.axis_index("c")
    s = lax.axis_index("s")
    rbase = s * ROWS_PER_SUB
    stripe = pl.ds(rbase, ROWS_PER_SUB)

    pltpu.sync_copy(zrs.at[stripe], acc.at[stripe])
    if with_deg:
        pltpu.sync_copy(zrs_deg.at[stripe], deg_acc.at[stripe])
        pltpu.sync_copy(ones, ones_v)
    plsc.subcore_barrier()

    def iissue(k, bi):
        pltpu.async_copy(eidx.at[s, pl.ds(2 * k, 2)], ibuf[bi], isem[bi])

    def iwait(k, bi):
        pltpu.make_async_copy(eidx.at[s, pl.ds(2 * k, 2)], ibuf[bi],
                              isem[bi]).wait()

    def gissue(k, bi, b):
        @pl.when(c == 0)
        def _():
            pltpu.async_copy(tab0.at[ibuf[bi].at[0]], rows[b], gsem[b])

        @pl.when(c == 1)
        def _():
            pltpu.async_copy(tab1.at[ibuf[bi].at[0]], rows[b], gsem[b])

    def gwait(bi, b):
        pltpu.make_async_copy(tab0.at[ibuf[bi].at[0]], rows[b],
                              gsem[b]).wait()

    def sissue(bi, b):
        pltpu.async_copy(rows[b], acc.at[ibuf[bi].at[1]], ssem[b], add=True)

    def swait(bi, b):
        pltpu.make_async_copy(rows[b], acc.at[ibuf[bi].at[1]],
                              ssem[b]).wait()

    def dissue(bi):
        pltpu.async_copy(ones_v, deg_acc.at[ibuf[bi].at[1]], degsem,
                         add=True)

    def dwait(bi):
        pltpu.make_async_copy(ones_v, deg_acc.at[ibuf[bi].at[1]],
                              degsem).wait()

    for k in range(LI):
        iissue(k, k)
    for k in range(look):
        iwait(k, k)
        gissue(k, k, k)

    def outer(k8, carry):
        for u in range(NIDX):
            k = k8 * NIDX + u
            b = u % nbuf
            bi = u

            @pl.when(k + LI < n_loc)
            def _(k=k, bi2=(u + LI) % NIDX):
                iissue(k + LI, bi2)

            gwait(bi, b)
            sissue(bi, b)
            if with_deg:
                dissue(bi)

                @pl.when(k >= look)
                def _(bi2=(u - look) % NIDX):
                    dwait(bi2)

            bg = (b + look) % nbuf

            @pl.when(k >= look)
            def _(bi2=(u - look) % NIDX, bg=bg):
                swait(bi2, bg)

            @pl.when(k + look < n_loc)
            def _(k=k, bi2=(u + look) % NIDX, bg=bg):
                iwait(k + look, bi2)
                gissue(k + look, bi2, bg)
        return carry

    lax.fori_loop(0, n_loc // NIDX, outer, 0)
    for k in range(n_loc - look, n_loc):
        swait(k % NIDX, k % nbuf)
        if with_deg:
            dwait(k % NIDX)
    plsc.subcore_barrier()

    @pl.when(c == 0)
    def _():
        pltpu.sync_copy(acc.at[stripe], out0.at[stripe])
        if with_deg:
            pltpu.sync_copy(deg_acc.at[stripe], dout.at[stripe])

    @pl.when(c == 1)
    def _():
        pltpu.sync_copy(acc.at[stripe], out1.at[stripe])


def _make_sc_agg(dc, blk, nbuf=4, look=2, with_deg=False, dt=jnp.bfloat16):
    mesh = plsc.VectorSubcoreMesh(core_axis_name="c", subcore_axis_name="s")
    out_type = [jax.ShapeDtypeStruct((N_PAD, dc), dt),
                jax.ShapeDtypeStruct((N_PAD, dc), dt)]
    scratch = (
        [pltpu.VMEM((2, blk), jnp.int32)] * NIDX
        + [pltpu.VMEM((blk, dc), dt)] * nbuf
        + [pltpu.VMEM_SHARED((N_PAD, dc), dt)]
        + [pltpu.SemaphoreType.DMA] * (NIDX + 2 * nbuf)
    )
    if with_deg:
        out_type += [jax.ShapeDtypeStruct((N_PAD, DEG_W), jnp.float32)]
        scratch += [
            pltpu.SemaphoreType.DMA,
            pltpu.VMEM((blk, DEG_W), jnp.float32),
            pltpu.VMEM_SHARED((N_PAD, DEG_W), jnp.float32),
        ]
    return pl.kernel(
        functools.partial(_sc_agg_body, with_deg, dc, blk, nbuf, look),
        out_type=out_type,
        mesh=mesh,
        scratch_types=scratch,
        compiler_params=pltpu.CompilerParams(use_tc_tiling_on_sc=False),
    )


def _pack_eidx(src_p, dst_p, blk):
    nb = E_PER_SUB // blk
    e = jnp.stack([src_p.reshape(NS, nb, blk),
                   dst_p.reshape(NS, nb, blk)], axis=2)
    return e.reshape(NS, 2 * nb, blk)


def _rdeg(deg_ref):
    return 1.0 / jnp.maximum(deg_ref[:N_NODES, 0:1], 1.0)


def _bn_relu(h, g_ref, be_ref):
    mu = jnp.mean(h, axis=0, keepdims=True)
    var = jnp.mean((h - mu) * (h - mu), axis=0, keepdims=True)
    h = g_ref[...] * (h - mu) * lax.rsqrt(var + EPS) + be_ref[...]
    return jnp.maximum(h, 0.0)


def _tc1_body(x_ref, a0_ref, a1_ref, d_ref,
              ws_ref, wn_ref, b_ref, g_ref, be_ref,
              h0_ref, h1_ref):
    rdeg = _rdeg(d_ref)
    f32 = jnp.float32
    h = jnp.dot(x_ref[...], ws_ref[...], preferred_element_type=f32)
    h += jnp.dot(a0_ref[:N_NODES].astype(f32) * rdeg, wn_ref[0],
                 preferred_element_type=f32)
    h += jnp.dot(a1_ref[:N_NODES].astype(f32) * rdeg, wn_ref[1],
                 preferred_element_type=f32)
    h += b_ref[...]
    h = _bn_relu(h, g_ref, be_ref)
    h0_ref[...] = h[:, :D_HID // 2].astype(h0_ref.dtype)
    h1_ref[...] = h[:, D_HID // 2:].astype(h1_ref.dtype)


def _tc2_body(x0_ref, x1_ref, a0_ref, a1_ref, d_ref,
              ws_ref, wn_ref, b_ref, g_ref, be_ref, wn3_ref,
              h0_ref, h1_ref, p_ref):
    rdeg = _rdeg(d_ref)
    f32 = jnp.float32
    h = jnp.dot(x0_ref[...], ws_ref[0], preferred_element_type=f32)
    h += jnp.dot(x1_ref[...], ws_ref[1], preferred_element_type=f32)
    h += jnp.dot(a0_ref[:N_NODES].astype(f32) * rdeg, wn_ref[0],
                 preferred_element_type=f32)
    h += jnp.dot(a1_ref[:N_NODES].astype(f32) * rdeg, wn_ref[1],
                 preferred_element_type=f32)
    h += b_ref[...]
    h = _bn_relu(h, g_ref, be_ref)
    h0_ref[...] = h[:, :D_HID // 2].astype(h0_ref.dtype)
    h1_ref[...] = h[:, D_HID // 2:].astype(h1_ref.dtype)
    p_ref[...] = jnp.dot(h, wn3_ref[...],
                         preferred_element_type=f32).astype(p_ref.dtype)


def _tc3_body(x0_ref, x1_ref, a0_ref, a1_ref, d_ref,
              ws_ref, b_ref, out_ref):
    rdeg = _rdeg(d_ref)
    f32 = jnp.float32
    mean = jnp.concatenate(
        [a0_ref[:N_NODES].astype(f32),
         a1_ref[:N_NODES, :N_CLS - 32].astype(f32)], axis=1) * rdeg
    logits = jnp.dot(x0_ref[...], ws_ref[0], preferred_element_type=f32)
    logits += jnp.dot(x1_ref[...], ws_ref[1], preferred_element_type=f32)
    logits += mean + b_ref[...]
    m = jnp.max(logits, axis=1, keepdims=True)
    sh = logits - m
    lse = jnp.log(jnp.sum(jnp.exp(sh), axis=1, keepdims=True))
    out_ref[...] = sh - lse


def kernel(x, edge_index,
           W_self1, W_neigh1, b1, gamma1, beta1,
           W_self2, W_neigh2, b2, gamma2, beta2,
           W_self3, W_neigh3, b3):
    f32 = jnp.float32
    bf16 = jnp.bfloat16
    src = edge_index[0].astype(jnp.int32)
    dst = edge_index[1].astype(jnp.int32)
    pad = E_PAD - N_EDGES
    src_p = jnp.concatenate([src, jnp.zeros((pad,), jnp.int32)])
    pad_dst = N_NODES + jnp.arange(pad, dtype=jnp.int32) % (N_PAD - N_NODES)
    dst_p = jnp.concatenate([dst, pad_dst])
    eidx128 = _pack_eidx(src_p, dst_p, 128)

    xb = x.astype(bf16)
    x0 = xb[:, :D_IN // 2]
    x1 = xb[:, D_IN // 2:]
    z64 = jnp.zeros((N_PAD, 64), bf16)
    z128 = jnp.zeros((N_PAD, 128), bf16)
    z32 = jnp.zeros((N_PAD, 32), bf16)
    zdeg = jnp.zeros((N_PAD, DEG_W), f32)
    ones = jnp.ones((128, DEG_W), f32)

    agg1_0, agg1_1, deg = _make_sc_agg(64, 128, with_deg=True)(
        eidx128, x0, x1, z64, zdeg, ones)

    wn1 = W_neigh1.reshape(2, 64, D_HID)
    b1r, g1r, be1r = b1.reshape(1, -1), gamma1.reshape(1, -1), beta1.reshape(1, -1)
    h1_0, h1_1 = pl.pallas_call(
        _tc1_body,
        out_shape=[jax.ShapeDtypeStruct((N_NODES, D_HID // 2), bf16)] * 2,
    )(x, agg1_0, agg1_1, deg, W_self1, wn1, b1r, g1r, be1r)

    agg2_0, agg2_1 = _make_sc_agg(128, 128)(eidx128, h1_0, h1_1, z128)

    wn2 = W_neigh2.reshape(2, 128, D_HID)
    wn3p = jnp.pad(W_neigh3, ((0, 0), (0, 64 - N_CLS)))
    b2r, g2r, be2r = b2.reshape(1, -1), gamma2.reshape(1, -1), beta2.reshape(1, -1)
    ws2 = W_self2.reshape(2, 128, D_HID)
    h2_0, h2_1, p3 = pl.pallas_call(
        _tc2_body,
        out_shape=[jax.ShapeDtypeStruct((N_NODES, D_HID // 2), bf16)] * 2
        + [jax.ShapeDtypeStruct((N_NODES, 64), bf16)],
    )(h1_0, h1_1, agg2_0, agg2_1, deg, ws2, wn2, b2r, g2r, be2r, wn3p)
    p3_0 = p3[:, :32]
    p3_1 = p3[:, 32:]

    agg3_0, agg3_1 = _make_sc_agg(32, 128)(eidx128, p3_0, p3_1, z32)

    b3r = b3.reshape(1, -1)
    ws3 = W_self3.reshape(2, 128, N_CLS)
    out = pl.pallas_call(
        _tc3_body,
        out_shape=jax.ShapeDtypeStruct((N_NODES, N_CLS), f32),
    )(h2_0, h2_1, agg3_0, agg3_1, deg, ws3, b3r)
    return out


# speedup vs baseline: 1.6414x; 1.0857x over previous
import functools

import jax
import jax.numpy as jnp
from jax import lax
from jax.experimental import pallas as pl
from jax.experimental.pallas import tpu as pltpu
from jax.experimental.pallas import tpu_sc as plsc

N_NODES = 10000
N_EDGES = 320000
D_IN = 128
D_HID = 256
N_CLS = 40
EPS = 1e-5

NC = 2
NS = 16
DEG_W = 8

N_PAD = 10112
E_PAD = 327680
E_PER_SUB = E_PAD // NS
ROWS_PER_SUB = N_PAD // NS

NIDX = 8
LI = 4


def _sc_agg_body(with_deg, dc, blk, nbuf, look, *refs):
    if with_deg:
        (eidx, tab0, tab1, zrs, zrs_deg, ones,
         out0, out1, dout, *rest) = refs
    else:
        (eidx, tab0, tab1, zrs, out0, out1, *rest) = refs
    ibuf = rest[:NIDX]
    rows = rest[NIDX:NIDX + nbuf]
    acc = rest[NIDX + nbuf]
    isem = rest[NIDX + nbuf + 1:NIDX + nbuf + 1 + NIDX]
    gsem = rest[NIDX + nbuf + 1 + NIDX:NIDX + nbuf + 1 + NIDX + nbuf]
    ssem = rest[NIDX + nbuf + 1 + NIDX + nbuf:
                NIDX + nbuf + 1 + NIDX + 2 * nbuf]
    if with_deg:
        degsem, ones_v, deg_acc = rest[NIDX + nbuf + 1 + NIDX + 2 * nbuf:]

    n_loc = E_PER_SUB // blk

    c = lax.axis_index("c")
    s = lax.axis_index("s")
    rbase = s * ROWS_PER_SUB
    stripe = pl.ds(rbase, ROWS_PER_SUB)

    pltpu.sync_copy(zrs.at[stripe], acc.at[stripe])
    if with_deg:
        pltpu.sync_copy(zrs_deg.at[stripe], deg_acc.at[stripe])
        pltpu.sync_copy(ones, ones_v)
    plsc.subcore_barrier()

    def iissue(k, bi):
        pltpu.async_copy(eidx.at[s, pl.ds(2 * k, 2)], ibuf[bi], isem[bi])

    def iwait(k, bi):
        pltpu.make_async_copy(eidx.at[s, pl.ds(2 * k, 2)], ibuf[bi],
                              isem[bi]).wait()

    def gissue(k, bi, b):
        @pl.when(c == 0)
        def _():
            pltpu.async_copy(tab0.at[ibuf[bi].at[0]], rows[b], gsem[b])

        @pl.when(c == 1)
        def _():
            pltpu.async_copy(tab1.at[ibuf[bi].at[0]], rows[b], gsem[b])

    def gwait(bi, b):
        pltpu.make_async_copy(tab0.at[ibuf[bi].at[0]], rows[b],
                              gsem[b]).wait()

    def sissue(bi, b):
        pltpu.async_copy(rows[b], acc.at[ibuf[bi].at[1]], ssem[b], add=True)

    def swait(bi, b):
        pltpu.make_async_copy(rows[b], acc.at[ibuf[bi].at[1]],
                              ssem[b]).wait()

    def dissue(bi):
        pltpu.async_copy(ones_v, deg_acc.at[ibuf[bi].at[1]], degsem,
                         add=True)

    def dwait(bi):
        pltpu.make_async_copy(ones_v, deg_acc.at[ibuf[bi].at[1]],
                              degsem).wait()

    for k in range(LI):
        iissue(k, k)
    for k in range(look):
        iwait(k, k)
        gissue(k, k, k)

    def outer(k8, carry):
        for u in range(NIDX):
            k = k8 * NIDX + u
            b = u % nbuf
            bi = u

            @pl.when(k + LI < n_loc)
            def _(k=k, bi2=(u + LI) % NIDX):
                iissue(k + LI, bi2)

            gwait(bi, b)
            sissue(bi, b)
            if with_deg:
                dissue(bi)

                @pl.when(k >= look)
                def _(bi2=(u - look) % NIDX):
                    dwait(bi2)

            bg = (b + look) % nbuf

            @pl.when(k >= look)
            def _(bi2=(u - look) % NIDX, bg=bg):
                swait(bi2, bg)

            @pl.when(k + look < n_loc)
            def _(k=k, bi2=(u + look) % NIDX, bg=bg):
                iwait(k + look, bi2)
                gissue(k + look, bi2, bg)
        return carry

    lax.fori_loop(0, n_loc // NIDX, outer, 0)
    for k in range(n_loc - look, n_loc):
        swait(k % NIDX, k % nbuf)
        if with_deg:
            dwait(k % NIDX)
    plsc.subcore_barrier()

    @pl.when(c == 0)
    def _():
        pltpu.sync_copy(acc.at[stripe], out0.at[stripe])
        if with_deg:
            pltpu.sync_copy(deg_acc.at[stripe], dout.at[stripe])

    @pl.when(c == 1)
    def _():
        pltpu.sync_copy(acc.at[stripe], out1.at[stripe])


def _make_sc_agg(dc, blk, nbuf=4, look=2, with_deg=False, dt=jnp.bfloat16):
    mesh = plsc.VectorSubcoreMesh(core_axis_name="c", subcore_axis_name="s")
    out_type = [jax.ShapeDtypeStruct((N_PAD, dc), dt),
                jax.ShapeDtypeStruct((N_PAD, dc), dt)]
    scratch = (
        [pltpu.VMEM((2, blk), jnp.int32)] * NIDX
        + [pltpu.VMEM((blk, dc), dt)] * nbuf
        + [pltpu.VMEM_SHARED((N_PAD, dc), dt)]
        + [pltpu.SemaphoreType.DMA] * (NIDX + 2 * nbuf)
    )
    if with_deg:
        out_type += [jax.ShapeDtypeStruct((N_PAD, DEG_W), jnp.float32)]
        scratch += [
            pltpu.SemaphoreType.DMA,
            pltpu.VMEM((blk, DEG_W), jnp.float32),
            pltpu.VMEM_SHARED((N_PAD, DEG_W), jnp.float32),
        ]
    return pl.kernel(
        functools.partial(_sc_agg_body, with_deg, dc, blk, nbuf, look),
        out_type=out_type,
        mesh=mesh,
        scratch_types=scratch,
        compiler_params=pltpu.CompilerParams(use_tc_tiling_on_sc=False),
    )


def _pack_eidx(src_p, dst_p, blk):
    nb = E_PER_SUB // blk
    e = jnp.stack([src_p.reshape(NS, nb, blk),
                   dst_p.reshape(NS, nb, blk)], axis=2)
    return e.reshape(NS, 2 * nb, blk)


def _rdeg(deg_ref):
    return 1.0 / jnp.maximum(deg_ref[:N_NODES, 0:1], 1.0)


def _bn_relu(h, g_ref, be_ref):
    mu = jnp.mean(h, axis=0, keepdims=True)
    var = jnp.mean((h - mu) * (h - mu), axis=0, keepdims=True)
    h = g_ref[...] * (h - mu) * lax.rsqrt(var + EPS) + be_ref[...]
    return jnp.maximum(h, 0.0)


def _scaled(a_ref, rdeg):
    bf16 = jnp.bfloat16
    return (a_ref[:N_NODES].astype(jnp.float32) * rdeg).astype(bf16)


def _tc1_body(x_ref, a0_ref, a1_ref, d_ref,
              ws_ref, wn_ref, b_ref, g_ref, be_ref,
              h0_ref, h1_ref):
    rdeg = _rdeg(d_ref)
    f32 = jnp.float32
    h = jnp.dot(x_ref[...], ws_ref[...], preferred_element_type=f32)
    h += jnp.dot(_scaled(a0_ref, rdeg), wn_ref[0],
                 preferred_element_type=f32)
    h += jnp.dot(_scaled(a1_ref, rdeg), wn_ref[1],
                 preferred_element_type=f32)
    h += b_ref[...]
    h = _bn_relu(h, g_ref, be_ref)
    h0_ref[...] = h[:, :D_HID // 2].astype(h0_ref.dtype)
    h1_ref[...] = h[:, D_HID // 2:].astype(h1_ref.dtype)


def _tc2_body(x0_ref, x1_ref, a0_ref, a1_ref, d_ref,
              ws_ref, wn_ref, b_ref, g_ref, be_ref, wn3_ref,
              h0_ref, h1_ref, p_ref):
    rdeg = _rdeg(d_ref)
    f32 = jnp.float32
    h = jnp.dot(x0_ref[...], ws_ref[0], preferred_element_type=f32)
    h += jnp.dot(x1_ref[...], ws_ref[1], preferred_element_type=f32)
    h += jnp.dot(_scaled(a0_ref, rdeg), wn_ref[0],
                 preferred_element_type=f32)
    h += jnp.dot(_scaled(a1_ref, rdeg), wn_ref[1],
                 preferred_element_type=f32)
    h += b_ref[...]
    h = _bn_relu(h, g_ref, be_ref)
    hb0 = h[:, :D_HID // 2].astype(h0_ref.dtype)
    hb1 = h[:, D_HID // 2:].astype(h1_ref.dtype)
    h0_ref[...] = hb0
    h1_ref[...] = hb1
    p = jnp.dot(hb0, wn3_ref[0], preferred_element_type=f32)
    p += jnp.dot(hb1, wn3_ref[1], preferred_element_type=f32)
    p_ref[...] = p.astype(p_ref.dtype)


def _tc3_body(x0_ref, x1_ref, a0_ref, a1_ref, d_ref,
              ws_ref, b_ref, out_ref):
    rdeg = _rdeg(d_ref)
    f32 = jnp.float32
    mean = jnp.concatenate(
        [a0_ref[:N_NODES].astype(f32),
         a1_ref[:N_NODES, :N_CLS - 32].astype(f32)], axis=1) * rdeg
    logits = jnp.dot(x0_ref[...], ws_ref[0], preferred_element_type=f32)
    logits += jnp.dot(x1_ref[...], ws_ref[1], preferred_element_type=f32)
    logits += mean + b_ref[...]
    m = jnp.max(logits, axis=1, keepdims=True)
    sh = logits - m
    lse = jnp.log(jnp.sum(jnp.exp(sh), axis=1, keepdims=True))
    out_ref[...] = sh - lse


def kernel(x, edge_index,
           W_self1, W_neigh1, b1, gamma1, beta1,
           W_self2, W_neigh2, b2, gamma2, beta2,
           W_self3, W_neigh3, b3):
    f32 = jnp.float32
    bf16 = jnp.bfloat16
    src = edge_index[0].astype(jnp.int32)
    dst = edge_index[1].astype(jnp.int32)
    pad = E_PAD - N_EDGES
    src_p = jnp.concatenate([src, jnp.zeros((pad,), jnp.int32)])
    pad_dst = N_NODES + jnp.arange(pad, dtype=jnp.int32) % (N_PAD - N_NODES)
    dst_p = jnp.concatenate([dst, pad_dst])
    eidx128 = _pack_eidx(src_p, dst_p, 128)

    xb = x.astype(bf16)
    x0 = xb[:, :D_IN // 2]
    x1 = xb[:, D_IN // 2:]
    z64 = jnp.zeros((N_PAD, 64), bf16)
    z128 = jnp.zeros((N_PAD, 128), bf16)
    z32 = jnp.zeros((N_PAD, 32), bf16)
    zdeg = jnp.zeros((N_PAD, DEG_W), f32)
    ones = jnp.ones((128, DEG_W), f32)

    agg1_0, agg1_1, deg = _make_sc_agg(64, 128, with_deg=True)(
        eidx128, x0, x1, z64, zdeg, ones)

    wn1 = W_neigh1.reshape(2, 64, D_HID).astype(bf16)
    b1r, g1r, be1r = b1.reshape(1, -1), gamma1.reshape(1, -1), beta1.reshape(1, -1)
    h1_0, h1_1 = pl.pallas_call(
        _tc1_body,
        out_shape=[jax.ShapeDtypeStruct((N_NODES, D_HID // 2), bf16)] * 2,
    )(xb, agg1_0, agg1_1, deg, W_self1.astype(bf16), wn1, b1r, g1r, be1r)

    agg2_0, agg2_1 = _make_sc_agg(128, 128)(eidx128, h1_0, h1_1, z128)

    wn2 = W_neigh2.reshape(2, 128, D_HID).astype(bf16)
    wn3p = jnp.pad(W_neigh3, ((0, 0), (0, 64 - N_CLS)))
    wn3p = wn3p.reshape(2, 128, 64).astype(bf16)
    b2r, g2r, be2r = b2.reshape(1, -1), gamma2.reshape(1, -1), beta2.reshape(1, -1)
    ws2 = W_self2.reshape(2, 128, D_HID).astype(bf16)
    h2_0, h2_1, p3 = pl.pallas_call(
        _tc2_body,
        out_shape=[jax.ShapeDtypeStruct((N_NODES, D_HID // 2), bf16)] * 2
        + [jax.ShapeDtypeStruct((N_NODES, 64), bf16)],
    )(h1_0, h1_1, agg2_0, agg2_1, deg, ws2, wn2, b2r, g2r, be2r, wn3p)
    p3_0 = p3[:, :32]
    p3_1 = p3[:, 32:]

    agg3_0, agg3_1 = _make_sc_agg(32, 128)(eidx128, p3_0, p3_1, z32)

    b3r = b3.reshape(1, -1)
    ws3 = W_self3.reshape(2, 128, N_CLS).astype(bf16)
    out = pl.pallas_call(
        _tc3_body,
        out_shape=jax.ShapeDtypeStruct((N_NODES, N_CLS), f32),
    )(h2_0, h2_1, agg3_0, agg3_1, deg, ws3, b3r)
    return out
